# baseline (speedup 1.0000x reference)
import functools

import jax
import jax.numpy as jnp
from jax.experimental import pallas as pl


def _bq_matmul_kernel(x_ref, binary_ref, scale_ref, bias_ref, out_ref):
    nbits, g, nf = binary_ref.shape
    pos = [scale_ref[b] for b in range(nbits)]
    neg = [-s for s in pos]
    slabs = []
    for i in range(8):
        acc = None
        for b in range(nbits):
            hit = (binary_ref[b] & (1 << (7 - i))) != 0
            fb = jnp.where(hit, pos[b], neg[b])
            acc = fb if acc is None else acc + fb
        slabs.append(acc)
    weff = jnp.stack(slabs, axis=0).reshape(8 * g, nf)
    out = jnp.dot(x_ref[...], weff.astype(jnp.bfloat16),
                  preferred_element_type=jnp.float32)
    out_ref[...] = out + bias_ref[...]


@functools.partial(jax.jit, static_argnames=())
def kernel(x, binary, scale, bias):
    size_out = x.shape[:-1] + (bias.shape[-1],)
    x2 = x.reshape(-1, x.shape[-1])
    t, nx = x2.shape
    nbits = scale.shape[1]
    nf = scale.shape[2]
    g = nx // 8
    xp = x2.reshape(t, g, 8).transpose(0, 2, 1).reshape(t, nx)
    xp = xp.astype(jnp.bfloat16)
    binary3 = binary.reshape(nbits, g, nf)
    scale3 = scale.reshape(nbits, 1, nf)
    bias2 = bias.reshape(1, nf)
    out = pl.pallas_call(
        _bq_matmul_kernel,
        out_shape=jax.ShapeDtypeStruct((t, nf), jnp.float32),
    )(xp, binary3, scale3, bias2)
    return out.reshape(size_out)


# speedup vs baseline: 1.3722x; 1.3722x over previous
import functools

import jax
import jax.numpy as jnp
from jax.experimental import pallas as pl


def _bq_matmul_kernel(x_ref, binary_ref, scale_ref, bias_ref, out_ref):
    nbits, g, nf = binary_ref.shape
    combo = binary_ref[0]
    for b in range(1, nbits):
        combo = combo | (binary_ref[b] << (8 * b))
    combo = jnp.broadcast_to(combo[:, None, :], (g, 8, nf))
    ii = jax.lax.broadcasted_iota(jnp.int32, (1, 8, 1), 1)
    w = None
    for b in range(nbits):
        mask = jnp.left_shift(1, 8 * b + 7 - ii)
        pos = scale_ref[b]
        fb = jnp.where((combo & mask) != 0, pos[None], -pos[None])
        w = fb if w is None else w + fb
    weff = w.reshape(g * 8, nf).astype(jnp.bfloat16)
    xb = x_ref[...].astype(jnp.bfloat16)
    out = jnp.dot(xb, weff, preferred_element_type=jnp.float32)
    out_ref[...] = out + bias_ref[...]


@functools.partial(jax.jit, static_argnames=())
def kernel(x, binary, scale, bias):
    size_out = x.shape[:-1] + (bias.shape[-1],)
    x2 = x.reshape(-1, x.shape[-1])
    t, nx = x2.shape
    nbits = scale.shape[1]
    nf = scale.shape[2]
    g = nx // 8
    binary3 = binary.reshape(nbits, g, nf)
    scale3 = scale.reshape(nbits, 1, nf)
    bias2 = bias.reshape(1, nf)
    out = pl.pallas_call(
        _bq_matmul_kernel,
        out_shape=jax.ShapeDtypeStruct((t, nf), jnp.float32),
    )(x2, binary3, scale3, bias2)
    return out.reshape(size_out)
